# baseline (speedup 1.0000x reference)
import functools

import jax
import jax.numpy as jnp
from jax import lax
from jax.experimental import pallas as pl
from jax.experimental.pallas import tpu as pltpu
from jax.experimental.pallas import tpu_sc as plsc

H = 4096
I = 11008
K = I // 2
IB = 512
NI = (I + IB - 1) // IB
HB = 512
NH = H // HB

NC = 2
NS = 16
LANES = 16


def _up_body(x_ref, wg_ref, wu_ref, prod_ref, bits_ref):
    x = x_ref[...]
    z = jax.lax.dot_general(x, wg_ref[...], (((1,), (1,)), ((), ())),
                            preferred_element_type=jnp.float32)
    u = jax.lax.dot_general(x, wu_ref[...], (((1,), (1,)), ((), ())),
                            preferred_element_type=jnp.float32)
    gate = z * (1.0 / (1.0 + jnp.exp(-z)))
    prod_ref[...] = gate * u
    bits_ref[...] = jax.lax.bitcast_convert_type(jnp.abs(gate), jnp.int32)


def _sc_select_body(bits_hbm, thr_hbm, row_v, hist_v, buf_a, buf_b, out_v):
    wid = lax.axis_index("s") * NC + lax.axis_index("c")
    pltpu.sync_copy(bits_hbm.at[wid], row_v)

    lane = lax.iota(jnp.int32, LANES)
    ones = jnp.ones((LANES,), jnp.int32)

    def round_select(src, n, k, shift):
        def zero(i, _):
            hist_v[pl.ds(i * LANES, LANES)] = jnp.zeros((LANES,), jnp.int32)
            return 0
        lax.fori_loop(0, 256, zero, 0)

        niter = (n + LANES - 1) // LANES

        def hist(i, _):
            v = src[pl.ds(i * LANES, LANES)]
            b = ((v >> shift) & 0xFF) * LANES + lane
            plsc.addupdate_scatter(hist_v, [b], ones, mask=lane < n - i * LANES)
            return 0
        lax.fori_loop(0, niter, hist, 0)

        def scan(e, carry):
            cum, e_sel, n_in, cumb = carry
            c = jnp.sum(hist_v[pl.ds(e * LANES, LANES)])
            hit = jnp.logical_and(cum + c >= k, e_sel < 0)
            e_sel = jnp.where(hit, e, e_sel)
            n_in = jnp.where(hit, c, n_in)
            cumb = jnp.where(hit, cum, cumb)
            return (cum + c, e_sel, n_in, cumb)
        _, e_sel, n_in, cumb = lax.fori_loop(
            0, 256, scan, (jnp.int32(0), jnp.int32(-1), jnp.int32(0),
                           jnp.int32(0)))
        return e_sel, n_in, k - cumb, niter

    def compact(src, dst, niter, n, shift, e_sel):
        def body(i, off):
            v = src[pl.ds(i * LANES, LANES)]
            m = jnp.logical_and(((v >> shift) & 0xFF) == e_sel,
                                lane < n - i * LANES)
            plsc.store_compressed(dst.at[pl.ds(off, LANES)], v, mask=m)
            return off + jnp.sum(m.astype(jnp.int32))
        lax.fori_loop(0, niter, body, jnp.int32(0))

    n0 = jnp.int32(I)
    k0 = jnp.int32(K)
    e1, n1, k1, it0 = round_select(row_v, n0, k0, 23)
    compact(row_v, buf_a, it0, n0, 23, e1)
    e2, n2, k2, it1 = round_select(buf_a, n1, k1, 15)
    compact(buf_a, buf_b, it1, n1, 15, e2)
    e3, n3, k3, it2 = round_select(buf_b, n2, k2, 7)
    compact(buf_b, buf_a, it2, n2, 7, e3)
    e4, _, _, _ = round_select(buf_a, n3, k3, 0)

    v_sel = (e1 << 23) | (e2 << 15) | (e3 << 7) | e4
    out_v[...] = jnp.zeros((LANES,), jnp.int32) + v_sel
    pltpu.sync_copy(out_v, thr_hbm.at[wid])


def _sc_select(bits):
    kfn = pl.kernel(
        _sc_select_body,
        out_type=jax.ShapeDtypeStruct((32, LANES), jnp.int32),
        mesh=plsc.VectorSubcoreMesh(core_axis_name="c", subcore_axis_name="s",
                                    num_cores=NC, num_subcores=NS),
        scratch_types=[
            pltpu.VMEM((I,), jnp.int32),
            pltpu.VMEM((256 * LANES,), jnp.int32),
            pltpu.VMEM((I,), jnp.int32),
            pltpu.VMEM((I,), jnp.int32),
            pltpu.VMEM((LANES,), jnp.int32),
        ],
        compiler_params=pltpu.CompilerParams(needs_layout_passes=False),
    )
    return kfn(bits)


def _down_body(bits_ref, prod_ref, thr_ref, wd_ref, out_ref, masked_ref):
    i = pl.program_id(0)

    @pl.when(i == 0)
    def _():
        v = thr_ref[:, 0:1]
        masked_ref[...] = jnp.where(bits_ref[...] > v, prod_ref[...], 0.0)

    out_ref[...] = jax.lax.dot_general(masked_ref[...], wd_ref[...],
                                       (((1,), (1,)), ((), ())),
                                       preferred_element_type=jnp.float32)


def kernel(x, Wg, Wu, Wd):
    B = x.shape[0]
    x2 = x.reshape(B, H)

    prod, bits = pl.pallas_call(
        _up_body,
        grid=(NI,),
        in_specs=[
            pl.BlockSpec((B, H), lambda i: (0, 0)),
            pl.BlockSpec((IB, H), lambda i: (i, 0)),
            pl.BlockSpec((IB, H), lambda i: (i, 0)),
        ],
        out_specs=[
            pl.BlockSpec((B, IB), lambda i: (0, i)),
            pl.BlockSpec((B, IB), lambda i: (0, i)),
        ],
        out_shape=[
            jax.ShapeDtypeStruct((B, I), jnp.float32),
            jax.ShapeDtypeStruct((B, I), jnp.int32),
        ],
    )(x2, Wg, Wu)

    thr = _sc_select(bits)

    out = pl.pallas_call(
        _down_body,
        grid=(NH,),
        in_specs=[
            pl.BlockSpec((B, I), lambda i: (0, 0)),
            pl.BlockSpec((B, I), lambda i: (0, 0)),
            pl.BlockSpec((B, LANES), lambda i: (0, 0)),
            pl.BlockSpec((HB, I), lambda i: (i, 0)),
        ],
        out_specs=pl.BlockSpec((B, HB), lambda i: (0, i)),
        out_shape=jax.ShapeDtypeStruct((B, H), jnp.float32),
        scratch_shapes=[pltpu.VMEM((B, I), jnp.float32)],
    )(bits, prod, thr, Wd)

    return out.reshape(B, 1, H)


# speedup vs baseline: 1.0260x; 1.0260x over previous
import functools

import jax
import jax.numpy as jnp
from jax import lax
from jax.experimental import pallas as pl
from jax.experimental.pallas import tpu as pltpu
from jax.experimental.pallas import tpu_sc as plsc

H = 4096
I = 11008
K = I // 2
IB = 512
NI = (I + IB - 1) // IB
HB = 512
NH = H // HB

NC = 2
NS = 16
LANES = 16


def _up_body(x_ref, wg_ref, wu_ref, prod_ref, bits_ref):
    x = x_ref[...]
    z = jax.lax.dot_general(x, wg_ref[...], (((1,), (1,)), ((), ())),
                            preferred_element_type=jnp.float32)
    u = jax.lax.dot_general(x, wu_ref[...], (((1,), (1,)), ((), ())),
                            preferred_element_type=jnp.float32)
    gate = z * (1.0 / (1.0 + jnp.exp(-z)))
    prod_ref[...] = gate * u
    bits_ref[...] = jax.lax.bitcast_convert_type(jnp.abs(gate), jnp.int32)


def _sc_select_body(bits_hbm, thr_hbm, row_v, hist_v, buf_a, buf_b, out_v):
    wid = lax.axis_index("s") * NC + lax.axis_index("c")
    pltpu.sync_copy(bits_hbm.at[wid], row_v)

    lane = lax.iota(jnp.int32, LANES)
    ones = jnp.ones((LANES,), jnp.int32)

    def round_select(src, n, k, shift, static_n=None):
        def zero(i, _):
            hist_v[pl.ds(i * LANES, LANES)] = jnp.zeros((LANES,), jnp.int32)
            return 0
        lax.fori_loop(0, 256, zero, 0, unroll=16)

        def hist(i, _):
            v = src[pl.ds(i * LANES, LANES)]
            b = ((v >> shift) & 0xFF) * LANES + lane
            plsc.addupdate_scatter(hist_v, [b], ones, mask=lane < n - i * LANES)
            return 0

        if static_n is not None:
            niter = (static_n + LANES - 1) // LANES
            lax.fori_loop(0, niter, hist, 0, unroll=8)
        else:
            niter = (n + LANES - 1) // LANES
            lax.fori_loop(0, niter, hist, 0)

        def scan(e, carry):
            cum, e_sel, n_in, cumb = carry
            c = jnp.sum(hist_v[pl.ds(e * LANES, LANES)])
            hit = jnp.logical_and(cum + c >= k, e_sel < 0)
            e_sel = jnp.where(hit, e, e_sel)
            n_in = jnp.where(hit, c, n_in)
            cumb = jnp.where(hit, cum, cumb)
            return (cum + c, e_sel, n_in, cumb)
        _, e_sel, n_in, cumb = lax.fori_loop(
            0, 256, scan, (jnp.int32(0), jnp.int32(-1), jnp.int32(0),
                           jnp.int32(0)), unroll=8)
        return e_sel, n_in, k - cumb, niter

    def compact(src, dst, niter, n, shift, e_sel, unroll=1):
        def body(i, off):
            v = src[pl.ds(i * LANES, LANES)]
            m = jnp.logical_and(((v >> shift) & 0xFF) == e_sel,
                                lane < n - i * LANES)
            plsc.store_compressed(dst.at[pl.ds(off, LANES)], v, mask=m)
            return off + jnp.sum(m.astype(jnp.int32))
        lax.fori_loop(0, niter, body, jnp.int32(0), unroll=unroll)

    n0 = jnp.int32(I)
    k0 = jnp.int32(K)
    e1, n1, k1, it0 = round_select(row_v, n0, k0, 23, static_n=I)
    compact(row_v, buf_a, it0, n0, 23, e1, unroll=8)
    e2, n2, k2, it1 = round_select(buf_a, n1, k1, 15)
    compact(buf_a, buf_b, it1, n1, 15, e2)
    e3, n3, k3, it2 = round_select(buf_b, n2, k2, 7)
    compact(buf_b, buf_a, it2, n2, 7, e3)
    e4, _, _, _ = round_select(buf_a, n3, k3, 0)

    v_sel = (e1 << 23) | (e2 << 15) | (e3 << 7) | e4
    out_v[...] = jnp.zeros((LANES,), jnp.int32) + v_sel
    pltpu.sync_copy(out_v, thr_hbm.at[wid])


def _sc_select(bits):
    kfn = pl.kernel(
        _sc_select_body,
        out_type=jax.ShapeDtypeStruct((32, LANES), jnp.int32),
        mesh=plsc.VectorSubcoreMesh(core_axis_name="c", subcore_axis_name="s",
                                    num_cores=NC, num_subcores=NS),
        scratch_types=[
            pltpu.VMEM((I,), jnp.int32),
            pltpu.VMEM((256 * LANES,), jnp.int32),
            pltpu.VMEM((I,), jnp.int32),
            pltpu.VMEM((I,), jnp.int32),
            pltpu.VMEM((LANES,), jnp.int32),
        ],
        compiler_params=pltpu.CompilerParams(needs_layout_passes=False),
    )
    return kfn(bits)


def _down_body(bits_ref, prod_ref, thr_ref, wd_ref, out_ref, masked_ref):
    i = pl.program_id(0)

    @pl.when(i == 0)
    def _():
        v = thr_ref[:, 0:1]
        masked_ref[...] = jnp.where(bits_ref[...] > v, prod_ref[...], 0.0)

    out_ref[...] = jax.lax.dot_general(masked_ref[...], wd_ref[...],
                                       (((1,), (1,)), ((), ())),
                                       preferred_element_type=jnp.float32)


def kernel(x, Wg, Wu, Wd):
    B = x.shape[0]
    x2 = x.reshape(B, H)

    prod, bits = pl.pallas_call(
        _up_body,
        grid=(NI,),
        in_specs=[
            pl.BlockSpec((B, H), lambda i: (0, 0)),
            pl.BlockSpec((IB, H), lambda i: (i, 0)),
            pl.BlockSpec((IB, H), lambda i: (i, 0)),
        ],
        out_specs=[
            pl.BlockSpec((B, IB), lambda i: (0, i)),
            pl.BlockSpec((B, IB), lambda i: (0, i)),
        ],
        out_shape=[
            jax.ShapeDtypeStruct((B, I), jnp.float32),
            jax.ShapeDtypeStruct((B, I), jnp.int32),
        ],
    )(x2, Wg, Wu)

    thr = _sc_select(bits)

    out = pl.pallas_call(
        _down_body,
        grid=(NH,),
        in_specs=[
            pl.BlockSpec((B, I), lambda i: (0, 0)),
            pl.BlockSpec((B, I), lambda i: (0, 0)),
            pl.BlockSpec((B, LANES), lambda i: (0, 0)),
            pl.BlockSpec((HB, I), lambda i: (i, 0)),
        ],
        out_specs=pl.BlockSpec((B, HB), lambda i: (0, i)),
        out_shape=jax.ShapeDtypeStruct((B, H), jnp.float32),
        scratch_shapes=[pltpu.VMEM((B, I), jnp.float32)],
    )(bits, prod, thr, Wd)

    return out.reshape(B, 1, H)


# speedup vs baseline: 1.1330x; 1.1043x over previous
import functools

import jax
import jax.numpy as jnp
from jax import lax
from jax.experimental import pallas as pl
from jax.experimental.pallas import tpu as pltpu
from jax.experimental.pallas import tpu_sc as plsc

H = 4096
I = 11008
K = I // 2
IB = 512
NI = (I + IB - 1) // IB
HB = 512
NH = H // HB

NC = 2
NS = 16
LANES = 16


def _gate_body(x_ref, wg_ref, gate_ref, bits_ref):
    x = x_ref[...]
    z = jax.lax.dot_general(x, wg_ref[...], (((1,), (1,)), ((), ())),
                            preferred_element_type=jnp.float32)
    gate = z * (1.0 / (1.0 + jnp.exp(-z)))
    gate_ref[...] = gate
    bits_ref[...] = jax.lax.bitcast_convert_type(jnp.abs(gate), jnp.int32)


def _up_body(x_ref, wu_ref, gate_ref, prod_ref):
    x = x_ref[...]
    u = jax.lax.dot_general(x, wu_ref[...], (((1,), (1,)), ((), ())),
                            preferred_element_type=jnp.float32)
    prod_ref[...] = gate_ref[...] * u


def _sc_select_body(bits_hbm, thr_hbm, row_v, hist_v, buf_a, buf_b, out_v):
    wid = lax.axis_index("s") * NC + lax.axis_index("c")
    pltpu.sync_copy(bits_hbm.at[wid], row_v)

    lane = lax.iota(jnp.int32, LANES)
    ones = jnp.ones((LANES,), jnp.int32)

    def round_select(src, n, k, shift, static_n=None):
        def zero(i, _):
            hist_v[pl.ds(i * LANES, LANES)] = jnp.zeros((LANES,), jnp.int32)
            return 0
        lax.fori_loop(0, 256, zero, 0, unroll=16)

        def hist(i, _):
            v = src[pl.ds(i * LANES, LANES)]
            b = ((v >> shift) & 0xFF) * LANES + lane
            plsc.addupdate_scatter(hist_v, [b], ones, mask=lane < n - i * LANES)
            return 0

        if static_n is not None:
            niter = (static_n + LANES - 1) // LANES
            lax.fori_loop(0, niter, hist, 0, unroll=8)
        else:
            niter = (n + LANES - 1) // LANES
            lax.fori_loop(0, niter, hist, 0)

        def scan(e, carry):
            cum, e_sel, n_in, cumb = carry
            c = jnp.sum(hist_v[pl.ds(e * LANES, LANES)])
            hit = jnp.logical_and(cum + c >= k, e_sel < 0)
            e_sel = jnp.where(hit, e, e_sel)
            n_in = jnp.where(hit, c, n_in)
            cumb = jnp.where(hit, cum, cumb)
            return (cum + c, e_sel, n_in, cumb)
        _, e_sel, n_in, cumb = lax.fori_loop(
            0, 256, scan, (jnp.int32(0), jnp.int32(-1), jnp.int32(0),
                           jnp.int32(0)), unroll=8)
        return e_sel, n_in, k - cumb, niter

    def compact(src, dst, niter, n, shift, e_sel, unroll=1):
        def body(i, off):
            v = src[pl.ds(i * LANES, LANES)]
            m = jnp.logical_and(((v >> shift) & 0xFF) == e_sel,
                                lane < n - i * LANES)
            plsc.store_compressed(dst.at[pl.ds(off, LANES)], v, mask=m)
            return off + jnp.sum(m.astype(jnp.int32))
        lax.fori_loop(0, niter, body, jnp.int32(0), unroll=unroll)

    n0 = jnp.int32(I)
    k0 = jnp.int32(K)
    e1, n1, k1, it0 = round_select(row_v, n0, k0, 23, static_n=I)
    compact(row_v, buf_a, it0, n0, 23, e1, unroll=8)
    e2, n2, k2, it1 = round_select(buf_a, n1, k1, 15)
    compact(buf_a, buf_b, it1, n1, 15, e2)
    e3, n3, k3, it2 = round_select(buf_b, n2, k2, 7)
    compact(buf_b, buf_a, it2, n2, 7, e3)
    e4, _, _, _ = round_select(buf_a, n3, k3, 0)

    v_sel = (e1 << 23) | (e2 << 15) | (e3 << 7) | e4
    out_v[...] = jnp.zeros((LANES,), jnp.int32) + v_sel
    pltpu.sync_copy(out_v, thr_hbm.at[wid])


def _sc_select(bits):
    kfn = pl.kernel(
        _sc_select_body,
        out_type=jax.ShapeDtypeStruct((32, LANES), jnp.int32),
        mesh=plsc.VectorSubcoreMesh(core_axis_name="c", subcore_axis_name="s",
                                    num_cores=NC, num_subcores=NS),
        scratch_types=[
            pltpu.VMEM((I,), jnp.int32),
            pltpu.VMEM((256 * LANES,), jnp.int32),
            pltpu.VMEM((I,), jnp.int32),
            pltpu.VMEM((I,), jnp.int32),
            pltpu.VMEM((LANES,), jnp.int32),
        ],
        compiler_params=pltpu.CompilerParams(needs_layout_passes=False),
    )
    return kfn(bits)


def _down_body(bits_ref, prod_ref, thr_ref, wd_ref, out_ref, masked_ref):
    i = pl.program_id(0)

    @pl.when(i == 0)
    def _():
        v = thr_ref[:, 0:1]
        masked_ref[...] = jnp.where(bits_ref[...] > v, prod_ref[...], 0.0)

    out_ref[...] = jax.lax.dot_general(masked_ref[...], wd_ref[...],
                                       (((1,), (1,)), ((), ())),
                                       preferred_element_type=jnp.float32)


def kernel(x, Wg, Wu, Wd):
    B = x.shape[0]
    x2 = x.reshape(B, H)

    gate, bits = pl.pallas_call(
        _gate_body,
        grid=(NI,),
        in_specs=[
            pl.BlockSpec((B, H), lambda i: (0, 0)),
            pl.BlockSpec((IB, H), lambda i: (i, 0)),
        ],
        out_specs=[
            pl.BlockSpec((B, IB), lambda i: (0, i)),
            pl.BlockSpec((B, IB), lambda i: (0, i)),
        ],
        out_shape=[
            jax.ShapeDtypeStruct((B, I), jnp.float32),
            jax.ShapeDtypeStruct((B, I), jnp.int32),
        ],
    )(x2, Wg)

    thr = _sc_select(bits)

    prod = pl.pallas_call(
        _up_body,
        grid=(NI,),
        in_specs=[
            pl.BlockSpec((B, H), lambda i: (0, 0)),
            pl.BlockSpec((IB, H), lambda i: (i, 0)),
            pl.BlockSpec((B, IB), lambda i: (0, i)),
        ],
        out_specs=pl.BlockSpec((B, IB), lambda i: (0, i)),
        out_shape=jax.ShapeDtypeStruct((B, I), jnp.float32),
    )(x2, Wu, gate)

    out = pl.pallas_call(
        _down_body,
        grid=(NH,),
        in_specs=[
            pl.BlockSpec((B, I), lambda i: (0, 0)),
            pl.BlockSpec((B, I), lambda i: (0, 0)),
            pl.BlockSpec((B, LANES), lambda i: (0, 0)),
            pl.BlockSpec((HB, I), lambda i: (i, 0)),
        ],
        out_specs=pl.BlockSpec((B, HB), lambda i: (0, i)),
        out_shape=jax.ShapeDtypeStruct((B, H), jnp.float32),
        scratch_shapes=[pltpu.VMEM((B, I), jnp.float32)],
    )(bits, prod, thr, Wd)

    return out.reshape(B, 1, H)


# speedup vs baseline: 1.1347x; 1.0014x over previous
import functools

import jax
import jax.numpy as jnp
from jax import lax
from jax.experimental import pallas as pl
from jax.experimental.pallas import tpu as pltpu
from jax.experimental.pallas import tpu_sc as plsc

H = 4096
I = 11008
K = I // 2
IB = 512
NI = (I + IB - 1) // IB
HB = 512
NH = H // HB

NC = 2
NS = 16
LANES = 16


def _gate_body(x_ref, wg_ref, gate_ref, bits_ref):
    x = x_ref[...]
    z = jax.lax.dot_general(x, wg_ref[...], (((1,), (1,)), ((), ())),
                            preferred_element_type=jnp.float32)
    gate = z * (1.0 / (1.0 + jnp.exp(-z)))
    gate_ref[...] = gate
    bits_ref[...] = jax.lax.bitcast_convert_type(jnp.abs(gate), jnp.int32)


def _up_body(x_ref, wu_ref, gate_ref, prod_ref):
    x = x_ref[...]
    u = jax.lax.dot_general(x, wu_ref[...], (((1,), (1,)), ((), ())),
                            preferred_element_type=jnp.float32)
    prod_ref[...] = gate_ref[...] * u


def _sc_select_body(bits_hbm, thr_hbm, row_v, hist_v, buf_a, buf_b, out_v):
    wid = lax.axis_index("s") * NC + lax.axis_index("c")
    pltpu.sync_copy(bits_hbm.at[wid], row_v)

    lane = lax.iota(jnp.int32, LANES)
    ones = jnp.ones((LANES,), jnp.int32)

    def round_select(src, n, k, shift, static_n=None):
        def zero(i, _):
            hist_v[pl.ds(i * LANES, LANES)] = jnp.zeros((LANES,), jnp.int32)
            return 0
        lax.fori_loop(0, 256, zero, 0, unroll=16)

        def hist(i, _):
            v = src[pl.ds(i * LANES, LANES)]
            b = ((v >> shift) & 0xFF) * LANES + lane
            plsc.addupdate_scatter(hist_v, [b], ones, mask=lane < n - i * LANES)
            return 0

        if static_n is not None:
            niter = (static_n + LANES - 1) // LANES
            lax.fori_loop(0, niter, hist, 0, unroll=8)
        else:
            niter = (n + LANES - 1) // LANES
            lax.fori_loop(0, niter, hist, 0)

        def scan(e, carry):
            cum, e_sel, n_in, cumb = carry
            c = jnp.sum(hist_v[pl.ds(e * LANES, LANES)])
            hit = jnp.logical_and(cum + c >= k, e_sel < 0)
            e_sel = jnp.where(hit, e, e_sel)
            n_in = jnp.where(hit, c, n_in)
            cumb = jnp.where(hit, cum, cumb)
            return (cum + c, e_sel, n_in, cumb)
        _, e_sel, n_in, cumb = lax.fori_loop(
            0, 256, scan, (jnp.int32(0), jnp.int32(-1), jnp.int32(0),
                           jnp.int32(0)), unroll=8)
        return e_sel, n_in, k - cumb, niter

    def compact(src, dst, niter, n, shift, e_sel, unroll=1):
        def body(i, off):
            v = src[pl.ds(i * LANES, LANES)]
            m = jnp.logical_and(((v >> shift) & 0xFF) == e_sel,
                                lane < n - i * LANES)
            plsc.store_compressed(dst.at[pl.ds(off, LANES)], v, mask=m)
            return off + jnp.sum(m.astype(jnp.int32))
        lax.fori_loop(0, niter, body, jnp.int32(0), unroll=unroll)

    n0 = jnp.int32(I)
    k0 = jnp.int32(K)
    e1, n1, k1, it0 = round_select(row_v, n0, k0, 23, static_n=I)
    compact(row_v, buf_a, it0, n0, 23, e1, unroll=8)
    e2, n2, k2, it1 = round_select(buf_a, n1, k1, 15)
    compact(buf_a, buf_b, it1, n1, 15, e2)
    e3, n3, k3, it2 = round_select(buf_b, n2, k2, 7)
    compact(buf_b, buf_a, it2, n2, 7, e3)
    e4, _, _, _ = round_select(buf_a, n3, k3, 0)

    v_sel = (e1 << 23) | (e2 << 15) | (e3 << 7) | e4
    out_v[...] = jnp.zeros((LANES,), jnp.int32) + v_sel
    pltpu.sync_copy(out_v, thr_hbm.at[wid])


def _sc_select(bits):
    kfn = pl.kernel(
        _sc_select_body,
        out_type=jax.ShapeDtypeStruct((32, LANES), jnp.int32),
        mesh=plsc.VectorSubcoreMesh(core_axis_name="c", subcore_axis_name="s",
                                    num_cores=NC, num_subcores=NS),
        scratch_types=[
            pltpu.VMEM((I,), jnp.int32),
            pltpu.VMEM((256 * LANES,), jnp.int32),
            pltpu.VMEM((I,), jnp.int32),
            pltpu.VMEM((I,), jnp.int32),
            pltpu.VMEM((LANES,), jnp.int32),
        ],
        compiler_params=pltpu.CompilerParams(needs_layout_passes=False, skip_device_barrier=True),
    )
    return kfn(bits)


def _down_body(bits_ref, prod_ref, thr_ref, wd_ref, out_ref, masked_ref):
    i = pl.program_id(0)

    @pl.when(i == 0)
    def _():
        v = thr_ref[:, 0:1]
        masked_ref[...] = jnp.where(bits_ref[...] > v, prod_ref[...], 0.0)

    out_ref[...] = jax.lax.dot_general(masked_ref[...], wd_ref[...],
                                       (((1,), (1,)), ((), ())),
                                       preferred_element_type=jnp.float32)


def kernel(x, Wg, Wu, Wd):
    B = x.shape[0]
    x2 = x.reshape(B, H)

    gate, bits = pl.pallas_call(
        _gate_body,
        grid=(NI,),
        in_specs=[
            pl.BlockSpec((B, H), lambda i: (0, 0)),
            pl.BlockSpec((IB, H), lambda i: (i, 0)),
        ],
        out_specs=[
            pl.BlockSpec((B, IB), lambda i: (0, i)),
            pl.BlockSpec((B, IB), lambda i: (0, i)),
        ],
        out_shape=[
            jax.ShapeDtypeStruct((B, I), jnp.float32),
            jax.ShapeDtypeStruct((B, I), jnp.int32),
        ],
    )(x2, Wg)

    thr = _sc_select(bits)

    prod = pl.pallas_call(
        _up_body,
        grid=(NI,),
        in_specs=[
            pl.BlockSpec((B, H), lambda i: (0, 0)),
            pl.BlockSpec((IB, H), lambda i: (i, 0)),
            pl.BlockSpec((B, IB), lambda i: (0, i)),
        ],
        out_specs=pl.BlockSpec((B, IB), lambda i: (0, i)),
        out_shape=jax.ShapeDtypeStruct((B, I), jnp.float32),
    )(x2, Wu, gate)

    out = pl.pallas_call(
        _down_body,
        grid=(NH,),
        in_specs=[
            pl.BlockSpec((B, I), lambda i: (0, 0)),
            pl.BlockSpec((B, I), lambda i: (0, 0)),
            pl.BlockSpec((B, LANES), lambda i: (0, 0)),
            pl.BlockSpec((HB, I), lambda i: (i, 0)),
        ],
        out_specs=pl.BlockSpec((B, HB), lambda i: (0, i)),
        out_shape=jax.ShapeDtypeStruct((B, H), jnp.float32),
        scratch_shapes=[pltpu.VMEM((B, I), jnp.float32)],
    )(bits, prod, thr, Wd)

    return out.reshape(B, 1, H)
